# baseline (device time: 47290 ns/iter reference)
import jax
import jax.numpy as jnp
from jax import lax
from jax.experimental import pallas as pl
from jax.experimental.pallas import tpu as pltpu

N_DEV = 4
B, SQ, SKV = 2, 256, 256
HQ, DH = 16, 64
H_LOC = HQ // N_DEV
D_MODEL = 512
WINDOW = 128


def kernel(x, Wq, K_ext, V_ext, Wo):
    my = lax.axis_index("i")
    K_h = jnp.moveaxis(K_ext, 2, 0)
    V_h = jnp.moveaxis(V_ext, 2, 0)
    k_mine = lax.dynamic_slice_in_dim(K_h, my * H_LOC, H_LOC, axis=0)
    v_mine = lax.dynamic_slice_in_dim(V_h, my * H_LOC, H_LOC, axis=0)

    def body(x_ref, wq_ref, k_ref, v_ref, wo_ref, out_ref,
             comm_ref, send_sems, recv_sems):
        my_pos = lax.axis_index("i")
        left = lax.rem(my_pos + (N_DEV - 1), N_DEV)
        right = lax.rem(my_pos + 1, N_DEV)

        barrier_sem = pltpu.get_barrier_semaphore()
        for nbr in (left, right):
            pl.semaphore_signal(barrier_sem, inc=1, device_id=(nbr,),
                                device_id_type=pl.DeviceIdType.MESH)
        pl.semaphore_wait(barrier_sem, 2)

        qi = lax.broadcasted_iota(jnp.int32, (SQ, SKV), 0)
        ki = lax.broadcasted_iota(jnp.int32, (SQ, SKV), 1)
        mask = jnp.abs(qi - ki) <= WINDOW

        for b in range(B):
            q = jnp.dot(x_ref[b], wq_ref[...],
                        preferred_element_type=jnp.float32)
            ctx_parts = []
            for h in range(H_LOC):
                qh = q[:, h * DH:(h + 1) * DH]
                kh = k_ref[h, b]
                vh = v_ref[h, b]
                s = lax.dot_general(
                    qh, kh, (((1,), (1,)), ((), ())),
                    preferred_element_type=jnp.float32) * 0.125
                s = jnp.where(mask, s, -1e9)
                s = s - jnp.max(s, axis=1, keepdims=True)
                w = jnp.exp(s)
                w = w / jnp.sum(w, axis=1, keepdims=True)
                ctx_parts.append(
                    jnp.dot(w, vh, preferred_element_type=jnp.float32))
            ctx = jnp.concatenate(ctx_parts, axis=1)
            part = jnp.dot(ctx, wo_ref[...],
                           preferred_element_type=jnp.float32)
            out_ref[b] = part
            comm_ref[0, b] = part

        for h in range(N_DEV - 1):
            rdma = pltpu.make_async_remote_copy(
                src_ref=comm_ref.at[h],
                dst_ref=comm_ref.at[h + 1],
                send_sem=send_sems.at[h],
                recv_sem=recv_sems.at[h],
                device_id=(right,),
                device_id_type=pl.DeviceIdType.MESH,
            )
            rdma.start()
            rdma.wait()
            out_ref[...] += comm_ref[h + 1]

    return pl.pallas_call(
        body,
        out_shape=jax.ShapeDtypeStruct((B, SQ, D_MODEL), jnp.float32),
        in_specs=[pl.BlockSpec(memory_space=pltpu.VMEM)] * 5,
        out_specs=pl.BlockSpec(memory_space=pltpu.VMEM),
        scratch_shapes=[
            pltpu.VMEM((N_DEV, B, SQ, D_MODEL), jnp.float32),
            pltpu.SemaphoreType.DMA((N_DEV - 1,)),
            pltpu.SemaphoreType.DMA((N_DEV - 1,)),
        ],
        compiler_params=pltpu.CompilerParams(collective_id=0),
    )(x, Wq, k_mine, v_mine, Wo)


# device time: 23907 ns/iter; 1.9781x vs baseline; 1.9781x over previous
import jax
import jax.numpy as jnp
from jax import lax
from jax.experimental import pallas as pl
from jax.experimental.pallas import tpu as pltpu

N_DEV = 4
B, SQ, SKV = 2, 256, 256
HQ, DH = 16, 64
H_LOC = HQ // N_DEV
D_MODEL = 512
HALF = D_MODEL // 2
WINDOW = 128


def kernel(x, Wq, K_ext, V_ext, Wo):
    my = lax.axis_index("i")
    K_h = jnp.moveaxis(K_ext, 2, 0)
    V_h = jnp.moveaxis(V_ext, 2, 0)
    k_mine = lax.dynamic_slice_in_dim(K_h, my * H_LOC, H_LOC, axis=0)
    v_mine = lax.dynamic_slice_in_dim(V_h, my * H_LOC, H_LOC, axis=0)

    def body(x_ref, wq_ref, k_ref, v_ref, wo_ref, out_ref,
             commA_ref, commB_ref, sA, rA, sB, rB):
        my_pos = lax.axis_index("i")
        x_partner = 3 - my_pos
        y_partner = my_pos ^ 1

        barrier_sem = pltpu.get_barrier_semaphore()
        for nbr in (x_partner, y_partner):
            pl.semaphore_signal(barrier_sem, inc=1, device_id=(nbr,),
                                device_id_type=pl.DeviceIdType.MESH)
        pl.semaphore_wait(barrier_sem, 2)

        qi = lax.broadcasted_iota(jnp.int32, (SQ, SKV), 0)
        ki = lax.broadcasted_iota(jnp.int32, (SQ, SKV), 1)
        mask = jnp.abs(qi - ki) <= WINDOW

        for b in range(B):
            q = jnp.dot(x_ref[b], wq_ref[...],
                        preferred_element_type=jnp.float32)
            ctx_parts = []
            for h in range(H_LOC):
                qh = q[:, h * DH:(h + 1) * DH]
                kh = k_ref[h, b]
                vh = v_ref[h, b]
                s = lax.dot_general(
                    qh, kh, (((1,), (1,)), ((), ())),
                    preferred_element_type=jnp.float32) * 0.125
                s = jnp.where(mask, s, -1e9)
                s = s - jnp.max(s, axis=1, keepdims=True)
                w = jnp.exp(s)
                w = w / jnp.sum(w, axis=1, keepdims=True)
                ctx_parts.append(
                    jnp.dot(w, vh, preferred_element_type=jnp.float32))
            ctx = jnp.concatenate(ctx_parts, axis=1)
            part = jnp.dot(ctx, wo_ref[...],
                           preferred_element_type=jnp.float32)
            commA_ref[0, b] = part[:, :HALF]
            commB_ref[0, b] = part[:, HALF:]

        a1 = pltpu.make_async_remote_copy(
            src_ref=commA_ref.at[0], dst_ref=commA_ref.at[1],
            send_sem=sA.at[0], recv_sem=rA.at[0],
            device_id=(x_partner,), device_id_type=pl.DeviceIdType.MESH)
        b1 = pltpu.make_async_remote_copy(
            src_ref=commB_ref.at[0], dst_ref=commB_ref.at[1],
            send_sem=sB.at[0], recv_sem=rB.at[0],
            device_id=(y_partner,), device_id_type=pl.DeviceIdType.MESH)
        a1.start()
        b1.start()
        a1.wait()
        b1.wait()
        commA_ref[0] = commA_ref[0] + commA_ref[1]
        commB_ref[0] = commB_ref[0] + commB_ref[1]

        a2 = pltpu.make_async_remote_copy(
            src_ref=commA_ref.at[0], dst_ref=commA_ref.at[2],
            send_sem=sA.at[1], recv_sem=rA.at[1],
            device_id=(y_partner,), device_id_type=pl.DeviceIdType.MESH)
        b2 = pltpu.make_async_remote_copy(
            src_ref=commB_ref.at[0], dst_ref=commB_ref.at[2],
            send_sem=sB.at[1], recv_sem=rB.at[1],
            device_id=(x_partner,), device_id_type=pl.DeviceIdType.MESH)
        a2.start()
        b2.start()
        a2.wait()
        b2.wait()
        out_ref[:, :, :HALF] = commA_ref[0] + commA_ref[2]
        out_ref[:, :, HALF:] = commB_ref[0] + commB_ref[2]

    return pl.pallas_call(
        body,
        out_shape=jax.ShapeDtypeStruct((B, SQ, D_MODEL), jnp.float32),
        in_specs=[pl.BlockSpec(memory_space=pltpu.VMEM)] * 5,
        out_specs=pl.BlockSpec(memory_space=pltpu.VMEM),
        scratch_shapes=[
            pltpu.VMEM((3, B, SQ, HALF), jnp.float32),
            pltpu.VMEM((3, B, SQ, HALF), jnp.float32),
            pltpu.SemaphoreType.DMA((2,)),
            pltpu.SemaphoreType.DMA((2,)),
            pltpu.SemaphoreType.DMA((2,)),
            pltpu.SemaphoreType.DMA((2,)),
        ],
        compiler_params=pltpu.CompilerParams(collective_id=0),
    )(x, Wq, k_mine, v_mine, Wo)


# device time: 20167 ns/iter; 2.3449x vs baseline; 1.1855x over previous
import jax
import jax.numpy as jnp
from jax import lax
from jax.experimental import pallas as pl
from jax.experimental.pallas import tpu as pltpu

N_DEV = 4
B, SQ, SKV = 2, 256, 256
HQ, DH = 16, 64
H_LOC = HQ // N_DEV
D_MODEL = 512
HALF = D_MODEL // 2
WINDOW = 128


def kernel(x, Wq, K_ext, V_ext, Wo):
    my = lax.axis_index("i")
    K_h = jnp.moveaxis(K_ext, 2, 0)
    V_h = jnp.moveaxis(V_ext, 2, 0)
    k_mine = lax.dynamic_slice_in_dim(K_h, my * H_LOC, H_LOC, axis=0)
    v_mine = lax.dynamic_slice_in_dim(V_h, my * H_LOC, H_LOC, axis=0)

    def body(x_ref, wq_ref, k_ref, v_ref, wo_ref, out_ref,
             commA_ref, commB_ref, sA, rA, sB, rB):
        my_pos = lax.axis_index("i")
        x_partner = 3 - my_pos
        y_partner = my_pos ^ 1

        barrier_sem = pltpu.get_barrier_semaphore()
        for nbr in (x_partner, y_partner):
            pl.semaphore_signal(barrier_sem, inc=1, device_id=(nbr,),
                                device_id_type=pl.DeviceIdType.MESH)
        pl.semaphore_wait(barrier_sem, 2)

        qi = lax.broadcasted_iota(jnp.int32, (SQ, SKV), 0)
        ki = lax.broadcasted_iota(jnp.int32, (SQ, SKV), 1)
        mask = jnp.abs(qi - ki) <= WINDOW

        def mk(comm_ref, stage, b, partner):
            return pltpu.make_async_remote_copy(
                src_ref=comm_ref.at[0, b],
                dst_ref=comm_ref.at[stage + 1, b],
                send_sem=(sA if comm_ref is commA_ref else sB).at[stage, b],
                recv_sem=(rA if comm_ref is commA_ref else rB).at[stage, b],
                device_id=(partner,), device_id_type=pl.DeviceIdType.MESH)

        stage1 = {}
        for b in range(B):
            q = jnp.dot(x_ref[b], wq_ref[...],
                        preferred_element_type=jnp.float32)
            ctx_parts = []
            for h in range(H_LOC):
                qh = q[:, h * DH:(h + 1) * DH]
                kh = k_ref[h, b]
                vh = v_ref[h, b]
                s = lax.dot_general(
                    qh, kh, (((1,), (1,)), ((), ())),
                    preferred_element_type=jnp.float32) * 0.125
                w = jnp.where(mask, jnp.exp(s), 0.0)
                recip = 1.0 / jnp.sum(w, axis=1, keepdims=True)
                ctx_parts.append(
                    jnp.dot(w, vh, preferred_element_type=jnp.float32)
                    * recip)
            ctx = jnp.concatenate(ctx_parts, axis=1)
            pA = jnp.dot(ctx, wo_ref[:, :HALF],
                         preferred_element_type=jnp.float32)
            commA_ref[0, b] = pA
            a1 = mk(commA_ref, 0, b, x_partner)
            a1.start()
            pB = jnp.dot(ctx, wo_ref[:, HALF:],
                         preferred_element_type=jnp.float32)
            commB_ref[0, b] = pB
            b1 = mk(commB_ref, 0, b, y_partner)
            b1.start()
            stage1[b] = (a1, b1)

        stage2 = {}
        for b in range(B):
            a1, b1 = stage1[b]
            a1.wait()
            commA_ref[0, b] = commA_ref[0, b] + commA_ref[1, b]
            a2 = mk(commA_ref, 1, b, y_partner)
            a2.start()
            b1.wait()
            commB_ref[0, b] = commB_ref[0, b] + commB_ref[1, b]
            b2 = mk(commB_ref, 1, b, x_partner)
            b2.start()
            stage2[b] = (a2, b2)

        for b in range(B):
            a2, b2 = stage2[b]
            a2.wait()
            out_ref[b, :, :HALF] = commA_ref[0, b] + commA_ref[2, b]
            b2.wait()
            out_ref[b, :, HALF:] = commB_ref[0, b] + commB_ref[2, b]

    return pl.pallas_call(
        body,
        out_shape=jax.ShapeDtypeStruct((B, SQ, D_MODEL), jnp.float32),
        in_specs=[pl.BlockSpec(memory_space=pltpu.VMEM)] * 5,
        out_specs=pl.BlockSpec(memory_space=pltpu.VMEM),
        scratch_shapes=[
            pltpu.VMEM((3, B, SQ, HALF), jnp.float32),
            pltpu.VMEM((3, B, SQ, HALF), jnp.float32),
            pltpu.SemaphoreType.DMA((2, B)),
            pltpu.SemaphoreType.DMA((2, B)),
            pltpu.SemaphoreType.DMA((2, B)),
            pltpu.SemaphoreType.DMA((2, B)),
        ],
        compiler_params=pltpu.CompilerParams(collective_id=0),
    )(x, Wq, k_mine, v_mine, Wo)


# device time: 17383 ns/iter; 2.7205x vs baseline; 1.1602x over previous
import jax
import jax.numpy as jnp
from jax import lax
from jax.experimental import pallas as pl
from jax.experimental.pallas import tpu as pltpu

N_DEV = 4
B, SQ, SKV = 2, 256, 256
HQ, DH = 16, 64
H_LOC = HQ // N_DEV
D_MODEL = 512
HALF = D_MODEL // 2
WINDOW = 128
BF = jnp.bfloat16


def kernel(x, Wq, K_ext, V_ext, Wo):
    my = lax.axis_index("i")
    K_h = jnp.moveaxis(K_ext, 2, 0)
    V_h = jnp.moveaxis(V_ext, 2, 0)
    k_mine = lax.dynamic_slice_in_dim(K_h, my * H_LOC, H_LOC, axis=0).astype(BF)
    v_mine = lax.dynamic_slice_in_dim(V_h, my * H_LOC, H_LOC, axis=0).astype(BF)

    def body(x_ref, wq_ref, k_ref, v_ref, wo_ref, out_ref,
             accA_ref, accB_ref, cA_ref, cB_ref, sA, rA, sB, rB):
        my_pos = lax.axis_index("i")
        x_partner = 3 - my_pos
        y_partner = my_pos ^ 1

        barrier_sem = pltpu.get_barrier_semaphore()
        for nbr in (x_partner, y_partner):
            pl.semaphore_signal(barrier_sem, inc=1, device_id=(nbr,),
                                device_id_type=pl.DeviceIdType.MESH)
        pl.semaphore_wait(barrier_sem, 2)

        qi = lax.broadcasted_iota(jnp.int32, (SQ, SKV), 0)
        ki = lax.broadcasted_iota(jnp.int32, (SQ, SKV), 1)
        mask = jnp.abs(qi - ki) <= WINDOW

        def mk(c_ref, sems, rems, stage, b, partner):
            return pltpu.make_async_remote_copy(
                src_ref=c_ref.at[2 * stage, b],
                dst_ref=c_ref.at[2 * stage + 1, b],
                send_sem=sems.at[stage, b],
                recv_sem=rems.at[stage, b],
                device_id=(partner,), device_id_type=pl.DeviceIdType.MESH)

        stage1 = {}
        for b in range(B):
            q = jnp.dot(x_ref[b], wq_ref[...],
                        preferred_element_type=jnp.float32)
            ctx_parts = []
            for h in range(H_LOC):
                qh = q[:, h * DH:(h + 1) * DH].astype(BF)
                kh = k_ref[h, b]
                vh = v_ref[h, b]
                s = lax.dot_general(
                    qh, kh, (((1,), (1,)), ((), ())),
                    preferred_element_type=jnp.float32) * 0.125
                w = jnp.where(mask, jnp.exp(s), 0.0)
                recip = 1.0 / jnp.sum(w, axis=1, keepdims=True)
                ctx_parts.append(
                    jnp.dot(w.astype(BF), vh,
                            preferred_element_type=jnp.float32) * recip)
            ctx = jnp.concatenate(ctx_parts, axis=1).astype(BF)
            pA = jnp.dot(ctx, wo_ref[:, :HALF],
                         preferred_element_type=jnp.float32)
            accA_ref[b] = pA
            cA_ref[0, b] = pA.astype(BF)
            a1 = mk(cA_ref, sA, rA, 0, b, x_partner)
            a1.start()
            pB = jnp.dot(ctx, wo_ref[:, HALF:],
                         preferred_element_type=jnp.float32)
            accB_ref[b] = pB
            cB_ref[0, b] = pB.astype(BF)
            b1 = mk(cB_ref, sB, rB, 0, b, y_partner)
            b1.start()
            stage1[b] = (a1, b1)

        stage2 = {}
        for b in range(B):
            a1, b1 = stage1[b]
            a1.wait()
            accA_ref[b] = accA_ref[b] + cA_ref[1, b].astype(jnp.float32)
            cA_ref[2, b] = accA_ref[b].astype(BF)
            a2 = mk(cA_ref, sA, rA, 1, b, y_partner)
            a2.start()
            b1.wait()
            accB_ref[b] = accB_ref[b] + cB_ref[1, b].astype(jnp.float32)
            cB_ref[2, b] = accB_ref[b].astype(BF)
            b2 = mk(cB_ref, sB, rB, 1, b, x_partner)
            b2.start()
            stage2[b] = (a2, b2)

        for b in range(B):
            a2, b2 = stage2[b]
            a2.wait()
            out_ref[b, :, :HALF] = accA_ref[b] + cA_ref[3, b].astype(jnp.float32)
            b2.wait()
            out_ref[b, :, HALF:] = accB_ref[b] + cB_ref[3, b].astype(jnp.float32)

    return pl.pallas_call(
        body,
        out_shape=jax.ShapeDtypeStruct((B, SQ, D_MODEL), jnp.float32),
        in_specs=[pl.BlockSpec(memory_space=pltpu.VMEM)] * 5,
        out_specs=pl.BlockSpec(memory_space=pltpu.VMEM),
        scratch_shapes=[
            pltpu.VMEM((B, SQ, HALF), jnp.float32),
            pltpu.VMEM((B, SQ, HALF), jnp.float32),
            pltpu.VMEM((4, B, SQ, HALF), BF),
            pltpu.VMEM((4, B, SQ, HALF), BF),
            pltpu.SemaphoreType.DMA((2, B)),
            pltpu.SemaphoreType.DMA((2, B)),
            pltpu.SemaphoreType.DMA((2, B)),
            pltpu.SemaphoreType.DMA((2, B)),
        ],
        compiler_params=pltpu.CompilerParams(collective_id=0),
    )(x.astype(BF), Wq.astype(BF), k_mine, v_mine, Wo.astype(BF))
